# Initial kernel scaffold; baseline (speedup 1.0000x reference)
import jax
import jax.numpy as jnp
from jax.experimental import pallas as pl


def kernel(x, der_desc_wrt_pos):
    raise NotImplementedError("write your pallas kernel here")


# speedup vs baseline: 1120.3268x; 1120.3268x over previous
import jax
import jax.numpy as jnp
from jax.experimental import pallas as pl


def _body(xsel_ref, der_ref, out_ref):
    y = jnp.dot(der_ref[0], xsel_ref[0], preferred_element_type=jnp.float32)
    out_ref[0] = y * y


def kernel(x, der_desc_wrt_pos):
    B, A, D, T = der_desc_wrt_pos.shape
    der2 = der_desc_wrt_pos.reshape(B, A, D * T)
    eye = jnp.eye(T, dtype=x.dtype)
    xsel = (x[:, :, None, None] * eye[None, None]).reshape(B, D * T, T)
    out = pl.pallas_call(
        _body,
        grid=(B,),
        in_specs=[
            pl.BlockSpec((1, D * T, T), lambda b: (b, 0, 0)),
            pl.BlockSpec((1, A, D * T), lambda b: (b, 0, 0)),
        ],
        out_specs=pl.BlockSpec((1, A, T), lambda b: (b, 0, 0)),
        out_shape=jax.ShapeDtypeStruct((B, A, T), jnp.float32),
    )(xsel, der2)
    return out.reshape(B, A * T)


# speedup vs baseline: 1161.9927x; 1.0372x over previous
import jax
import jax.numpy as jnp
from jax import lax
from jax.experimental import pallas as pl


def _body(x_ref, der_ref, out_ref):
    y = lax.dot_general(
        der_ref[0], x_ref[0, 0],
        dimension_numbers=(((1,), (0,)), ((), ())),
        preferred_element_type=jnp.float32,
    )
    out_ref[0, 0] = y * y


def kernel(x, der_desc_wrt_pos):
    B, A, D, T = der_desc_wrt_pos.shape
    der_t = der_desc_wrt_pos.transpose(0, 1, 3, 2).reshape(B, A * T, D)
    x3 = x.reshape(B, 1, D)
    out = pl.pallas_call(
        _body,
        grid=(B,),
        in_specs=[
            pl.BlockSpec((1, 1, D), lambda b: (b, 0, 0)),
            pl.BlockSpec((1, A * T, D), lambda b: (b, 0, 0)),
        ],
        out_specs=pl.BlockSpec((1, 1, A * T), lambda b: (b, 0, 0)),
        out_shape=jax.ShapeDtypeStruct((B, 1, A * T), jnp.float32),
    )(x3, der_t)
    return out.reshape(B, A * T)


# speedup vs baseline: 4018.1595x; 3.4580x over previous
import jax
import jax.numpy as jnp
from jax import lax
from jax.experimental import pallas as pl


def _body(x_ref, der_ref, out_ref):
    xb = jnp.broadcast_to(x_ref[0], (8, x_ref.shape[-1]))
    y = lax.dot_general(
        xb, der_ref[0],
        dimension_numbers=(((1,), (1,)), ((), ())),
        preferred_element_type=jnp.float32,
    )
    y0 = y[0]
    out_ref[0, 0] = y0 * y0


def kernel(x, der_desc_wrt_pos):
    B, A, D, T = der_desc_wrt_pos.shape
    der_t = der_desc_wrt_pos.transpose(0, 3, 1, 2).reshape(B, T * A, D)
    x3 = x.reshape(B, 1, D)
    out = pl.pallas_call(
        _body,
        grid=(B,),
        in_specs=[
            pl.BlockSpec((1, 1, D), lambda b: (b, 0, 0)),
            pl.BlockSpec((1, T * A, D), lambda b: (b, 0, 0)),
        ],
        out_specs=pl.BlockSpec((1, 1, T * A), lambda b: (b, 0, 0)),
        out_shape=jax.ShapeDtypeStruct((B, 1, T * A), jnp.float32),
    )(x3, der_t)
    return out.reshape(B, T, A).transpose(0, 2, 1).reshape(B, A * T)


# speedup vs baseline: 8358.2401x; 2.0801x over previous
import jax
import jax.numpy as jnp
from jax import lax
from jax.experimental import pallas as pl

_BB = 8


def _body(x_ref, der_ref, out_ref):
    for i in range(_BB):
        xb = jnp.broadcast_to(x_ref[0, i], (8, x_ref.shape[-1]))
        y = lax.dot_general(
            xb, der_ref[i],
            dimension_numbers=(((1,), (1,)), ((), ())),
            preferred_element_type=jnp.float32,
        )
        y0 = y[0]
        out_ref[0, i] = y0 * y0


def kernel(x, der_desc_wrt_pos):
    B, A, D, T = der_desc_wrt_pos.shape
    der_t = der_desc_wrt_pos.transpose(0, 3, 1, 2).reshape(B, T * A, D)
    x3 = x.reshape(B // _BB, _BB, D)
    out = pl.pallas_call(
        _body,
        grid=(B // _BB,),
        in_specs=[
            pl.BlockSpec((1, _BB, D), lambda b: (b, 0, 0)),
            pl.BlockSpec((_BB, T * A, D), lambda b: (b, 0, 0)),
        ],
        out_specs=pl.BlockSpec((1, _BB, T * A), lambda b: (b, 0, 0)),
        out_shape=jax.ShapeDtypeStruct((B // _BB, _BB, T * A), jnp.float32),
    )(x3, der_t)
    return out.reshape(B, T, A).transpose(0, 2, 1).reshape(B, A * T)
